# baseline (speedup 1.0000x reference)
import numpy as np
import jax
import jax.numpy as jnp
from jax.experimental import pallas as pl

_B, _N, _T, _F = 8, 2048, 96, 16
_TO, _E, _K = 12, 2, 25
_NB = 512


def _avg_mat(t, k):
    p = (k - 1) // 2
    a = np.zeros((t, t), np.float32)
    for to in range(t):
        for j in range(to - p, to + p + 1):
            a[min(max(j, 0), t - 1), to] += 1.0 / k
    return a


def _down_mat(t):
    d = np.zeros((t, t // 2), np.float32)
    for i in range(t // 2):
        d[2 * i, i] = 0.5
        d[2 * i + 1, i] = 0.5
    return d


_D96 = _down_mat(96)
_DN = np.concatenate([_D96, _D96 @ _down_mat(48)], axis=1)
_A96 = _avg_mat(96, _K)
_A12 = np.zeros((72, 72), np.float32)
_A12[0:48, 0:48] = _avg_mat(48, _K)
_A12[48:72, 48:72] = _avg_mat(24, _K)
_S0M = np.eye(96, dtype=np.float32) - _A96
_S12 = np.eye(72, dtype=np.float32) - _A12
_R1 = np.concatenate([_S12[:, 0:48], _A12[:, 0:48], _A12[:, 48:72]], axis=1)
_R2A = np.zeros((72, 120), np.float32)
_R2A[:, 0:24] = _S12[:, 48:72]
_R2B = np.zeros((96, 120), np.float32)
_R2B[:, 24:120] = _A96
_INV_SQRT2 = np.float32(1.0 / np.sqrt(2.0))


def _gelu_pre(u):
    return u + u * jax.lax.erf(u)


def _bd(a, b):
    (ra, ca), (rb, cb) = a.shape, b.shape
    z = jnp.zeros((ra + rb, ca + cb), jnp.float32)
    return z.at[:ra, :ca].set(a).at[ra:, ca:].set(b)


def _body(x_ref, dn_ref, q1_ref, q2_ref, r1_ref, r2a_ref, r2b_ref, s0m_ref,
          b2_ref, c1_ref, c2_ref, b3_ref, c3_ref, b4_ref, c4_ref,
          wc1_ref, bc1_ref, wc2_ref, wc2b_ref, bc2_ref, out_ref):
    nb = x_ref.shape[0]
    m = nb * _F
    bf16 = jnp.bfloat16
    dotf = lambda a, w: jnp.dot(a, w, preferred_element_type=jnp.float32)
    x0 = x_ref[...].reshape(m, _T)
    x12 = dotf(x0, dn_ref[...]).astype(bf16)
    for b in range(_E):
        pre1 = dotf(x0, q1_ref[b]) + dotf(x12, q2_ref[b]) + c1_ref[b]
        g1 = _gelu_pre(pre1).astype(bf16)
        r1 = dotf(x12, r1_ref[...])
        v1 = dotf(g1, b2_ref[b]) + c2_ref[b] + r1[:, 0:96]
        v1b = v1.astype(bf16)
        g2 = _gelu_pre(dotf(v1b, b3_ref[b]) + c3_ref[b]).astype(bf16)
        r2 = dotf(x12, r2a_ref[...]) + dotf(x0, r2b_ref[...])
        v2 = dotf(g2, b4_ref[b]) + c4_ref[b] + r2
        s0 = dotf(x0, s0m_ref[...])
        x0 = (s0 + v2[:, 24:120]).astype(bf16)
        x12 = jnp.concatenate([v1[:, 0:48] + v1[:, 48:96],
                               v2[:, 0:24] + r1[:, 96:120]],
                              axis=1).astype(bf16)
    x0f = x0.reshape(nb, _F, _T).astype(jnp.float32)
    z = jnp.sum(x0f * wc2b_ref[...], axis=1)
    bhead = bc1_ref[...] * jnp.sum(wc2_ref[...]) + bc2_ref[0, 0]
    out_ref[...] = jnp.dot(z, wc1_ref[...]) + bhead


def kernel(x, sW1_0, sb1_0, sW2_0, sb2_0, sW1_1, sb1_1, sW2_1, sb2_1,
           tW1_0, tb1_0, tW2_0, tb2_0, tW1_1, tb1_1, tW2_1, tb2_1,
           Wc1, bc1, Wc2, bc2):
    xf = jnp.swapaxes(x.reshape(_B * _N, _T, _F), 1, 2).astype(jnp.bfloat16)
    stk = lambda f: jnp.stack([f(b) for b in range(_E)])
    rs2 = _INV_SQRT2
    b1 = stk(lambda b: _bd(sW1_0[b].T, tW1_1[b].T)) * rs2
    q1 = jnp.einsum('ij,bjk->bik', jnp.asarray(_S0M), b1[:, 0:96, :])
    q2 = jnp.einsum('ij,bjk->bik', jnp.asarray(_A12[:, 48:72]),
                    b1[:, 96:120, :])
    b2 = stk(lambda b: _bd(sW2_0[b].T, tW2_1[b].T)) * rs2
    b3 = stk(lambda b: _bd(sW1_1[b].T, tW1_0[b].T)) * rs2
    b4 = stk(lambda b: _bd(sW2_1[b].T, tW2_0[b].T)) * rs2
    cat = lambda u, v: jnp.concatenate([u, v], axis=1)[:, None, :]
    c1 = cat(sb1_0, tb1_1) * rs2
    c2 = cat(sb2_0, tb2_1)
    c3 = cat(sb1_1, tb1_0) * rs2
    c4 = cat(sb2_1, tb2_0)
    bf = lambda a: a.astype(jnp.bfloat16)
    ops = (
        jnp.asarray(_DN, jnp.bfloat16),
        bf(q1), bf(q2), jnp.asarray(_R1, jnp.bfloat16),
        jnp.asarray(_R2A, jnp.bfloat16), jnp.asarray(_R2B, jnp.bfloat16),
        jnp.asarray(_S0M, jnp.bfloat16),
        bf(b2), bf(c1), bf(c2), bf(b3), bf(c3), bf(b4), bf(c4),
        Wc1.T, bc1.reshape(1, _TO), Wc2,
        jnp.broadcast_to(Wc2.reshape(1, _F, 1), (1, _F, _T)),
        bc2.reshape(1, 1),
    )
    full = lambda a: pl.BlockSpec(a.shape, lambda i: (0,) * a.ndim)
    grid = (_B * _N // _NB,)
    out = pl.pallas_call(
        _body,
        grid=grid,
        in_specs=[pl.BlockSpec((_NB, _F, _T), lambda i: (i, 0, 0))]
                 + [full(a) for a in ops],
        out_specs=pl.BlockSpec((_NB, _TO), lambda i: (i, 0)),
        out_shape=jax.ShapeDtypeStruct((_B * _N, _TO), jnp.float32),
    )(xf, *ops)
    return out.reshape(_B, _N, _TO)


# speedup vs baseline: 1.0103x; 1.0103x over previous
import numpy as np
import jax
import jax.numpy as jnp
from jax.experimental import pallas as pl

_B, _N, _T, _F = 8, 2048, 96, 16
_TO, _E, _K = 12, 2, 25
_NB = 1024


def _avg_mat(t, k):
    p = (k - 1) // 2
    a = np.zeros((t, t), np.float32)
    for to in range(t):
        for j in range(to - p, to + p + 1):
            a[min(max(j, 0), t - 1), to] += 1.0 / k
    return a


def _down_mat(t):
    d = np.zeros((t, t // 2), np.float32)
    for i in range(t // 2):
        d[2 * i, i] = 0.5
        d[2 * i + 1, i] = 0.5
    return d


_D96 = _down_mat(96)
_DN = np.concatenate([_D96, _D96 @ _down_mat(48)], axis=1)
_A96 = _avg_mat(96, _K)
_A12 = np.zeros((72, 72), np.float32)
_A12[0:48, 0:48] = _avg_mat(48, _K)
_A12[48:72, 48:72] = _avg_mat(24, _K)
_S0M = np.eye(96, dtype=np.float32) - _A96
_S12 = np.eye(72, dtype=np.float32) - _A12
_R1 = np.concatenate([_S12[:, 0:48], _A12[:, 0:48], _A12[:, 48:72]], axis=1)
_R2A = np.zeros((72, 120), np.float32)
_R2A[:, 0:24] = _S12[:, 48:72]
_R2B = np.zeros((96, 120), np.float32)
_R2B[:, 24:120] = _A96
_INV_SQRT2 = np.float32(1.0 / np.sqrt(2.0))


def _gelu_pre(u):
    return u + u * jax.lax.erf(u)


def _bd(a, b):
    (ra, ca), (rb, cb) = a.shape, b.shape
    z = jnp.zeros((ra + rb, ca + cb), jnp.float32)
    return z.at[:ra, :ca].set(a).at[ra:, ca:].set(b)


def _body(x_ref, dn_ref, q1_ref, q2_ref, r1_ref, r2a_ref, r2b_ref, s0m_ref,
          b2_ref, c1_ref, c2_ref, b3_ref, c3_ref, b4_ref, c4_ref,
          wc1_ref, bc1_ref, wc2_ref, wc2b_ref, bc2_ref, out_ref):
    nb = x_ref.shape[0]
    m = nb * _F
    bf16 = jnp.bfloat16
    dotf = lambda a, w: jnp.dot(a, w, preferred_element_type=jnp.float32)
    x0 = x_ref[...].reshape(m, _T)
    x12 = dotf(x0, dn_ref[...]).astype(bf16)
    for b in range(_E):
        pre1 = dotf(x0, q1_ref[b]) + dotf(x12, q2_ref[b]) + c1_ref[b]
        g1 = _gelu_pre(pre1).astype(bf16)
        r1 = dotf(x12, r1_ref[...])
        v1 = dotf(g1, b2_ref[b]) + c2_ref[b] + r1[:, 0:96]
        v1b = v1.astype(bf16)
        g2 = _gelu_pre(dotf(v1b, b3_ref[b]) + c3_ref[b]).astype(bf16)
        r2 = dotf(x12, r2a_ref[...]) + dotf(x0, r2b_ref[...])
        v2 = dotf(g2, b4_ref[b]) + c4_ref[b] + r2
        s0 = dotf(x0, s0m_ref[...])
        x0 = (s0 + v2[:, 24:120]).astype(bf16)
        x12 = jnp.concatenate([v1[:, 0:48] + v1[:, 48:96],
                               v2[:, 0:24] + r1[:, 96:120]],
                              axis=1).astype(bf16)
    x0f = x0.reshape(nb, _F, _T).astype(jnp.float32)
    z = jnp.sum(x0f * wc2b_ref[...], axis=1)
    bhead = bc1_ref[...] * jnp.sum(wc2_ref[...]) + bc2_ref[0, 0]
    out_ref[...] = jnp.dot(z, wc1_ref[...]) + bhead


def kernel(x, sW1_0, sb1_0, sW2_0, sb2_0, sW1_1, sb1_1, sW2_1, sb2_1,
           tW1_0, tb1_0, tW2_0, tb2_0, tW1_1, tb1_1, tW2_1, tb2_1,
           Wc1, bc1, Wc2, bc2):
    xf = jnp.swapaxes(x.reshape(_B * _N, _T, _F), 1, 2).astype(jnp.bfloat16)
    stk = lambda f: jnp.stack([f(b) for b in range(_E)])
    rs2 = _INV_SQRT2
    b1 = stk(lambda b: _bd(sW1_0[b].T, tW1_1[b].T)) * rs2
    q1 = jnp.einsum('ij,bjk->bik', jnp.asarray(_S0M), b1[:, 0:96, :])
    q2 = jnp.einsum('ij,bjk->bik', jnp.asarray(_A12[:, 48:72]),
                    b1[:, 96:120, :])
    b2 = stk(lambda b: _bd(sW2_0[b].T, tW2_1[b].T)) * rs2
    b3 = stk(lambda b: _bd(sW1_1[b].T, tW1_0[b].T)) * rs2
    b4 = stk(lambda b: _bd(sW2_1[b].T, tW2_0[b].T)) * rs2
    cat = lambda u, v: jnp.concatenate([u, v], axis=1)[:, None, :]
    c1 = cat(sb1_0, tb1_1) * rs2
    c2 = cat(sb2_0, tb2_1)
    c3 = cat(sb1_1, tb1_0) * rs2
    c4 = cat(sb2_1, tb2_0)
    bf = lambda a: a.astype(jnp.bfloat16)
    ops = (
        jnp.asarray(_DN, jnp.bfloat16),
        bf(q1), bf(q2), jnp.asarray(_R1, jnp.bfloat16),
        jnp.asarray(_R2A, jnp.bfloat16), jnp.asarray(_R2B, jnp.bfloat16),
        jnp.asarray(_S0M, jnp.bfloat16),
        bf(b2), bf(c1), bf(c2), bf(b3), bf(c3), bf(b4), bf(c4),
        Wc1.T, bc1.reshape(1, _TO), Wc2,
        jnp.broadcast_to(Wc2.reshape(1, _F, 1), (1, _F, _T)),
        bc2.reshape(1, 1),
    )
    full = lambda a: pl.BlockSpec(a.shape, lambda i: (0,) * a.ndim)
    grid = (_B * _N // _NB,)
    out = pl.pallas_call(
        _body,
        grid=grid,
        in_specs=[pl.BlockSpec((_NB, _F, _T), lambda i: (i, 0, 0))]
                 + [full(a) for a in ops],
        out_specs=pl.BlockSpec((_NB, _TO), lambda i: (i, 0)),
        out_shape=jax.ShapeDtypeStruct((_B * _N, _TO), jnp.float32),
    )(xf, *ops)
    return out.reshape(_B, _N, _TO)


# speedup vs baseline: 1.1351x; 1.1235x over previous
import numpy as np
import jax
import jax.numpy as jnp
from jax.experimental import pallas as pl

_B, _N, _T, _F = 8, 2048, 96, 16
_TO, _E, _K = 12, 2, 25
_NB = 1024


def _avg_mat(t, k):
    p = (k - 1) // 2
    a = np.zeros((t, t), np.float32)
    for to in range(t):
        for j in range(to - p, to + p + 1):
            a[min(max(j, 0), t - 1), to] += 1.0 / k
    return a


def _down_mat(t):
    d = np.zeros((t, t // 2), np.float32)
    for i in range(t // 2):
        d[2 * i, i] = 0.5
        d[2 * i + 1, i] = 0.5
    return d


_D96 = _down_mat(96)
_DN = np.concatenate([_D96, _D96 @ _down_mat(48)], axis=1)
_A96 = _avg_mat(96, _K)
_A12 = np.zeros((72, 72), np.float32)
_A12[0:48, 0:48] = _avg_mat(48, _K)
_A12[48:72, 48:72] = _avg_mat(24, _K)
_INV_SQRT2 = np.float32(1.0 / np.sqrt(2.0))


def _gelu_pre(u):
    return u + u * jax.lax.erf(u)


def _bd(a, b):
    (ra, ca), (rb, cb) = a.shape, b.shape
    z = jnp.zeros((ra + rb, ca + cb), jnp.float32)
    return z.at[:ra, :ca].set(a).at[ra:, ca:].set(b)


def _body(x_ref, dn_ref, a96_ref, a12_ref,
          b1_ref, c1_ref, b2_ref, c2_ref,
          b3_ref, c3_ref, b4_ref, c4_ref,
          wc1_ref, bc1_ref, wc2_ref, wc2b_ref, bc2_ref, out_ref):
    nb = x_ref.shape[0]
    m = nb * _F
    bf16 = jnp.bfloat16
    dotf = lambda a, w: jnp.dot(a, w, preferred_element_type=jnp.float32)
    x0 = x_ref[...].reshape(m, _T)
    x12 = dotf(x0, dn_ref[...]).astype(bf16)
    for b in range(_E):
        m0 = dotf(x0, a96_ref[...]).astype(bf16)
        m12 = dotf(x12, a12_ref[...]).astype(bf16)
        s0 = x0 - m0
        s12 = x12 - m12
        u1 = jnp.concatenate([s0, m12[:, 48:72]], axis=1)
        g1 = _gelu_pre(dotf(u1, b1_ref[b]) + c1_ref[b]).astype(bf16)
        v1 = (dotf(g1, b2_ref[b]) + c2_ref[b]
              + jnp.concatenate([s12[:, 0:48], m12[:, 0:48]], axis=1)
              ).astype(bf16)
        g2 = _gelu_pre(dotf(v1, b3_ref[b]) + c3_ref[b]).astype(bf16)
        v2 = (dotf(g2, b4_ref[b]) + c4_ref[b]
              + jnp.concatenate([s12[:, 48:72], m0], axis=1)
              ).astype(bf16)
        x0 = s0 + v2[:, 24:120]
        x12 = jnp.concatenate([v1[:, 0:48] + v1[:, 48:96],
                               v2[:, 0:24] + m12[:, 48:72]], axis=1)
    x0f = x0.reshape(nb, _F, _T).astype(jnp.float32)
    z = jnp.sum(x0f * wc2b_ref[...], axis=1)
    bhead = bc1_ref[...] * jnp.sum(wc2_ref[...]) + bc2_ref[0, 0]
    out_ref[...] = jnp.dot(z, wc1_ref[...]) + bhead


def kernel(x, sW1_0, sb1_0, sW2_0, sb2_0, sW1_1, sb1_1, sW2_1, sb2_1,
           tW1_0, tb1_0, tW2_0, tb2_0, tW1_1, tb1_1, tW2_1, tb2_1,
           Wc1, bc1, Wc2, bc2):
    xf = jnp.swapaxes(x.reshape(_B * _N, _T, _F), 1, 2).astype(jnp.bfloat16)
    stk = lambda f: jnp.stack([f(b) for b in range(_E)])
    rs2 = _INV_SQRT2
    b1 = stk(lambda b: _bd(sW1_0[b].T, tW1_1[b].T)) * rs2
    b2 = stk(lambda b: _bd(sW2_0[b].T, tW2_1[b].T)) * rs2
    b3 = stk(lambda b: _bd(sW1_1[b].T, tW1_0[b].T)) * rs2
    b4 = stk(lambda b: _bd(sW2_1[b].T, tW2_0[b].T)) * rs2
    cat = lambda u, v: jnp.concatenate([u, v], axis=1)[:, None, :]
    c1 = cat(sb1_0, tb1_1) * rs2
    c2 = cat(sb2_0, tb2_1)
    c3 = cat(sb1_1, tb1_0) * rs2
    c4 = cat(sb2_1, tb2_0)
    bf = lambda a: a.astype(jnp.bfloat16)
    ops = (
        jnp.asarray(_DN, jnp.bfloat16), jnp.asarray(_A96, jnp.bfloat16),
        jnp.asarray(_A12, jnp.bfloat16),
        bf(b1), bf(c1), bf(b2), bf(c2), bf(b3), bf(c3), bf(b4), bf(c4),
        Wc1.T, bc1.reshape(1, _TO), Wc2,
        jnp.broadcast_to(Wc2.reshape(1, _F, 1), (1, _F, _T)),
        bc2.reshape(1, 1),
    )
    full = lambda a: pl.BlockSpec(a.shape, lambda i: (0,) * a.ndim)
    grid = (_B * _N // _NB,)
    out = pl.pallas_call(
        _body,
        grid=grid,
        in_specs=[pl.BlockSpec((_NB, _F, _T), lambda i: (i, 0, 0))]
                 + [full(a) for a in ops],
        out_specs=pl.BlockSpec((_NB, _TO), lambda i: (i, 0)),
        out_shape=jax.ShapeDtypeStruct((_B * _N, _TO), jnp.float32),
    )(xf, *ops)
    return out.reshape(_B, _N, _TO)


# speedup vs baseline: 1.2274x; 1.0813x over previous
import numpy as np
import jax
import jax.numpy as jnp
from jax.experimental import pallas as pl

_B, _N, _T, _F = 8, 2048, 96, 16
_TO, _E, _K = 12, 2, 25
_NB = 1024


def _avg_mat(t, k):
    p = (k - 1) // 2
    a = np.zeros((t, t), np.float32)
    for to in range(t):
        for j in range(to - p, to + p + 1):
            a[min(max(j, 0), t - 1), to] += 1.0 / k
    return a


def _down_mat(t):
    d = np.zeros((t, t // 2), np.float32)
    for i in range(t // 2):
        d[2 * i, i] = 0.5
        d[2 * i + 1, i] = 0.5
    return d


_D96 = _down_mat(96)
_DN = np.concatenate([_D96, _D96 @ _down_mat(48)], axis=1)
_A96 = _avg_mat(96, _K)
_A12 = np.zeros((72, 72), np.float32)
_A12[0:48, 0:48] = _avg_mat(48, _K)
_A12[48:72, 48:72] = _avg_mat(24, _K)
_INV_SQRT2 = np.float32(1.0 / np.sqrt(2.0))


def _gelu_pre(u):
    return u + u * jax.lax.erf(u)


def _bd(a, b):
    (ra, ca), (rb, cb) = a.shape, b.shape
    z = jnp.zeros((ra + rb, ca + cb), jnp.float32)
    return z.at[:ra, :ca].set(a).at[ra:, ca:].set(b)


def _body(x_ref, dn_ref, a96_ref, a12_ref,
          b1_ref, c1_ref, b2_ref, c2_ref,
          b3_ref, c3_ref, b4_ref, c4_ref,
          wc1_ref, bc1_ref, wc2_ref, wc2b_ref, bc2_ref, out_ref):
    nb = x_ref.shape[0]
    m = nb * _F
    bf16 = jnp.bfloat16
    dotf = lambda a, w: jnp.dot(a, w, preferred_element_type=jnp.float32)
    x0 = x_ref[...].reshape(m, _T)
    x12 = dotf(x0, dn_ref[...]).astype(bf16)
    for b in range(_E):
        m0 = dotf(x0, a96_ref[...]).astype(bf16)
        m12 = dotf(x12, a12_ref[...]).astype(bf16)
        s0 = x0 - m0
        s12 = x12 - m12
        u1 = jnp.concatenate([s0, m12[:, 48:72]], axis=1)
        g1 = _gelu_pre(dotf(u1, b1_ref[b]) + c1_ref[b]).astype(bf16)
        v1 = (dotf(g1, b2_ref[b]).astype(bf16) + c2_ref[b]
              + jnp.concatenate([s12[:, 0:48], m12[:, 0:48]], axis=1))
        g2 = _gelu_pre(dotf(v1, b3_ref[b]) + c3_ref[b]).astype(bf16)
        v2 = (dotf(g2, b4_ref[b]).astype(bf16) + c4_ref[b]
              + jnp.concatenate([s12[:, 48:72], m0], axis=1))
        x0 = s0 + v2[:, 24:120]
        x12 = jnp.concatenate([v1[:, 0:48] + v1[:, 48:96],
                               v2[:, 0:24] + m12[:, 48:72]], axis=1)
    x0f = x0.reshape(nb, _F, _T).astype(jnp.float32)
    z = jnp.sum(x0f * wc2b_ref[...], axis=1)
    bhead = bc1_ref[...] * jnp.sum(wc2_ref[...]) + bc2_ref[0, 0]
    out_ref[...] = jnp.dot(z, wc1_ref[...]) + bhead


def kernel(x, sW1_0, sb1_0, sW2_0, sb2_0, sW1_1, sb1_1, sW2_1, sb2_1,
           tW1_0, tb1_0, tW2_0, tb2_0, tW1_1, tb1_1, tW2_1, tb2_1,
           Wc1, bc1, Wc2, bc2):
    xf = jnp.swapaxes(x.reshape(_B * _N, _T, _F), 1, 2).astype(jnp.bfloat16)
    stk = lambda f: jnp.stack([f(b) for b in range(_E)])
    rs2 = _INV_SQRT2
    b1 = stk(lambda b: _bd(sW1_0[b].T, tW1_1[b].T)) * rs2
    b2 = stk(lambda b: _bd(sW2_0[b].T, tW2_1[b].T)) * rs2
    b3 = stk(lambda b: _bd(sW1_1[b].T, tW1_0[b].T)) * rs2
    b4 = stk(lambda b: _bd(sW2_1[b].T, tW2_0[b].T)) * rs2
    cat = lambda u, v: jnp.concatenate([u, v], axis=1)[:, None, :]
    c1 = cat(sb1_0, tb1_1) * rs2
    c2 = cat(sb2_0, tb2_1)
    c3 = cat(sb1_1, tb1_0) * rs2
    c4 = cat(sb2_1, tb2_0)
    bf = lambda a: a.astype(jnp.bfloat16)
    ops = (
        jnp.asarray(_DN, jnp.bfloat16), jnp.asarray(_A96, jnp.bfloat16),
        jnp.asarray(_A12, jnp.bfloat16),
        bf(b1), bf(c1), bf(b2), bf(c2), bf(b3), bf(c3), bf(b4), bf(c4),
        Wc1.T, bc1.reshape(1, _TO), Wc2,
        jnp.broadcast_to(Wc2.reshape(1, _F, 1), (1, _F, _T)),
        bc2.reshape(1, 1),
    )
    full = lambda a: pl.BlockSpec(a.shape, lambda i: (0,) * a.ndim)
    grid = (_B * _N // _NB,)
    out = pl.pallas_call(
        _body,
        grid=grid,
        in_specs=[pl.BlockSpec((_NB, _F, _T), lambda i: (i, 0, 0))]
                 + [full(a) for a in ops],
        out_specs=pl.BlockSpec((_NB, _TO), lambda i: (i, 0)),
        out_shape=jax.ShapeDtypeStruct((_B * _N, _TO), jnp.float32),
    )(xf, *ops)
    return out.reshape(_B, _N, _TO)
